# baseline (speedup 1.0000x reference)
import functools

import jax
import jax.numpy as jnp
import numpy as np
from jax import lax
from jax.experimental import pallas as pl
from jax.experimental.pallas import tpu as pltpu
from jax.experimental.pallas import tpu_sc as plsc

_C = 100000
_B = 1024
_SMOOTH = 0.1

_FILL = float(np.float32(_SMOOTH / (_C - 1)))
_ON = float(np.float32(1.0 - _SMOOTH))
_CONST = _B * ((_C - 1) * _FILL * np.log(_FILL) + _ON * np.log(_ON))
_INV_N = 1.0 / (_B * _C)
_K0 = np.float32(_CONST * _INV_N)
_K1 = np.float32(-_FILL * _INV_N)
_K2 = np.float32(-(_ON - _FILL) * _INV_N)

_NC, _NS, _NL = 2, 16, 16
_NW = _NC * _NS
_RPW = _B // _NW

_SCC = 16000
_SC_CHUNK = 32
_SC_NCHUNK = _SCC // _SC_CHUNK

_TC_BLK = 4000
_TC_GRID = (_C - _SCC) // _TC_BLK
_TC_OFF = _SCC // _TC_BLK


def _sc_body(xt, tgt, out, t_v, tiles_v, part_v, buf_v, sem):
    wid = lax.axis_index("s") * _NC + lax.axis_index("c")
    base = wid * _RPW
    pltpu.sync_copy(tgt.at[pl.ds(base, _RPW)], t_v)
    copies = []
    for h in range(_RPW // _NL):
        tv8 = t_v[pl.ds(h * _NL, _NL)] & -8
        for l in range(_NL):
            j = h * _NL + l
            rowb = pl.multiple_of(tv8[l], 8)
            colb = ((base + j) // 128) * 128
            copies.append(
                pltpu.async_copy(
                    xt.at[pl.ds(rowb, 8), pl.ds(colb, 128)],
                    tiles_v.at[j],
                    sem,
                )
            )
    for c in copies:
        c.wait()
    acc = None
    lane = lax.iota(jnp.int32, _NL)
    for h in range(_RPW // _NL):
        tv = t_v[pl.ds(h * _NL, _NL)]
        trow = tv & 7
        for l in range(_NL):
            j = h * _NL + l
            lb = ((base + j) % 128) & -_NL
            row16 = tiles_v[j, trow[l], pl.ds(lb, _NL)]
            sel = jnp.where(lane == (base + j) % _NL, row16, 0.0)
            acc = sel if acc is None else acc + sel
    part_v[...] = acc
    pltpu.sync_copy(part_v, out.at[0, wid])
    def chunk_fn(k, a):
        c = wid + k * _NW
        rb = pl.multiple_of(c * _SC_CHUNK, 8)
        pltpu.sync_copy(xt.at[pl.ds(rb, _SC_CHUNK)], buf_v)
        s = a
        for r in range(_SC_CHUNK):
            def inner(j, aa, r=r):
                t = aa
                for u in range(8):
                    t = t + buf_v[r, pl.ds(j * 128 + u * _NL, _NL)]
                return t
            s = lax.fori_loop(0, _B // 128, inner, s)
        return s

    nk = (_SC_NCHUNK - wid + _NW - 1) // _NW
    acc2 = lax.fori_loop(
        0, nk, chunk_fn, jnp.zeros((_NL,), jnp.float32)
    )
    part_v[...] = acc2
    pltpu.sync_copy(part_v, out.at[1, wid])


@functools.cache
def _sc_call():
    return functools.partial(
        pl.kernel,
        mesh=plsc.VectorSubcoreMesh(core_axis_name="c", subcore_axis_name="s"),
        out_type=jax.ShapeDtypeStruct((2, _NW, _NL), jnp.float32),
        scratch_types=[
            pltpu.VMEM((_RPW,), jnp.int32),
            pltpu.VMEM((_RPW, 8, 128), jnp.float32),
            pltpu.VMEM((_NL,), jnp.float32),
            pltpu.VMEM((_SC_CHUNK, _B), jnp.float32),
            pltpu.SemaphoreType.DMA,
        ],
    )(_sc_body)


def _tc_sum_body(x_ref, out_ref, acc_ref):
    i = pl.program_id(0)

    @pl.when(i == 0)
    def _init():
        acc_ref[0, 0] = 0.0

    acc_ref[0, 0] += jnp.sum(x_ref[...])

    @pl.when(i == _TC_GRID - 1)
    def _fin():
        out_ref[0, 0] = acc_ref[0, 0]


def _combine_body(s_ref, p_ref, out_ref):
    g = jnp.sum(p_ref[0])
    ssc = jnp.sum(p_ref[1])
    out_ref[0, 0] = _K0 + _K1 * (s_ref[0, 0] + ssc) + _K2 * g


def kernel(X, target):
    xt = X.T
    sc_parts = _sc_call()(xt, target)
    tc_sum = pl.pallas_call(
        _tc_sum_body,
        grid=(_TC_GRID,),
        in_specs=[pl.BlockSpec((_TC_BLK, _B), lambda i: (i + _TC_OFF, 0))],
        out_specs=pl.BlockSpec(
            (1, 1), lambda i: (0, 0), memory_space=pltpu.SMEM
        ),
        out_shape=jax.ShapeDtypeStruct((1, 1), jnp.float32),
        scratch_shapes=[pltpu.SMEM((1, 1), jnp.float32)],
    )(xt)
    out = pl.pallas_call(
        _combine_body,
        in_specs=[
            pl.BlockSpec(memory_space=pltpu.SMEM),
            pl.BlockSpec((2, _NW, _NL), lambda: (0, 0, 0)),
        ],
        out_specs=pl.BlockSpec(memory_space=pltpu.SMEM),
        out_shape=jax.ShapeDtypeStruct((1, 1), jnp.float32),
    )(tc_sum, sc_parts)
    return out.reshape(())


# speedup vs baseline: 1.0229x; 1.0229x over previous
import functools

import jax
import jax.numpy as jnp
import numpy as np
from jax import lax
from jax.experimental import pallas as pl
from jax.experimental.pallas import tpu as pltpu
from jax.experimental.pallas import tpu_sc as plsc

_C = 100000
_B = 1024
_SMOOTH = 0.1

_FILL = float(np.float32(_SMOOTH / (_C - 1)))
_ON = float(np.float32(1.0 - _SMOOTH))
_CONST = _B * ((_C - 1) * _FILL * np.log(_FILL) + _ON * np.log(_ON))
_INV_N = 1.0 / (_B * _C)
_K0 = np.float32(_CONST * _INV_N)
_K1 = np.float32(-_FILL * _INV_N)
_K2 = np.float32(-(_ON - _FILL) * _INV_N)

_NC, _NS, _NL = 2, 16, 16
_NW = _NC * _NS
_RPW = _B // _NW

_SCC = 16384
_SC_CHUNK = 32
_SC_CPW = _SCC // (_SC_CHUNK * _NW)
_SC_BASE = _C - _SCC

_TC_BLK = 3216
_TC_GRID = (_C - _SCC) // _TC_BLK
_TC_OFF = 0


def _sc_body(xt, tgt, out, t_v, tiles_v, part_v, bufa_v, bufb_v, sema, semb, sem):
    wid = lax.axis_index("s") * _NC + lax.axis_index("c")
    base = wid * _RPW
    pltpu.sync_copy(tgt.at[pl.ds(base, _RPW)], t_v)
    copies = []
    for h in range(_RPW // _NL):
        tv8 = t_v[pl.ds(h * _NL, _NL)] & -8
        for l in range(_NL):
            j = h * _NL + l
            rowb = pl.multiple_of(tv8[l], 8)
            colb = ((base + j) // 128) * 128
            copies.append(
                pltpu.async_copy(
                    xt.at[pl.ds(rowb, 8), pl.ds(colb, 128)],
                    tiles_v.at[j],
                    sem,
                )
            )
    for c in copies:
        c.wait()
    acc = None
    lane = lax.iota(jnp.int32, _NL)
    for h in range(_RPW // _NL):
        tv = t_v[pl.ds(h * _NL, _NL)]
        trow = tv & 7
        for l in range(_NL):
            j = h * _NL + l
            lb = ((base + j) % 128) & -_NL
            row16 = tiles_v[j, trow[l], pl.ds(lb, _NL)]
            sel = jnp.where(lane == (base + j) % _NL, row16, 0.0)
            acc = sel if acc is None else acc + sel
    part_v[...] = acc
    pltpu.sync_copy(part_v, out.at[0, wid])
    cbase = wid * _SC_CPW

    def start(k, buf, sem):
        rb = pl.multiple_of(_SC_BASE + (cbase + k) * _SC_CHUNK, 8)
        pltpu.make_async_copy(
            xt.at[pl.ds(rb, _SC_CHUNK)], buf, sem
        ).start()

    def wait(buf, sem):
        pltpu.make_async_copy(
            xt.at[pl.ds(0, _SC_CHUNK)], buf, sem
        ).wait()

    def bsum(buf, a):
        s = a
        for r in range(_SC_CHUNK):
            def inner(j, aa, r=r):
                t = aa
                for u in range(8):
                    t = t + buf[r, pl.ds(j * 128 + u * _NL, _NL)]
                return t
            s = lax.fori_loop(0, _B // 128, inner, s)
        return s

    start(0, bufa_v, sema)

    def pair(m, a):
        start(2 * m + 1, bufb_v, semb)
        wait(bufa_v, sema)
        a = bsum(bufa_v, a)

        @pl.when(m < _SC_CPW // 2 - 1)
        def _():
            start(2 * m + 2, bufa_v, sema)

        wait(bufb_v, semb)
        return bsum(bufb_v, a)

    acc2 = lax.fori_loop(
        0, _SC_CPW // 2, pair, jnp.zeros((_NL,), jnp.float32)
    )
    part_v[...] = acc2
    pltpu.sync_copy(part_v, out.at[1, wid])


@functools.cache
def _sc_call():
    return functools.partial(
        pl.kernel,
        mesh=plsc.VectorSubcoreMesh(core_axis_name="c", subcore_axis_name="s"),
        out_type=jax.ShapeDtypeStruct((2, _NW, _NL), jnp.float32),
        scratch_types=[
            pltpu.VMEM((_RPW,), jnp.int32),
            pltpu.VMEM((_RPW, 8, 128), jnp.float32),
            pltpu.VMEM((_NL,), jnp.float32),
            pltpu.VMEM((_SC_CHUNK, _B), jnp.float32),
            pltpu.VMEM((_SC_CHUNK, _B), jnp.float32),
            pltpu.SemaphoreType.DMA,
            pltpu.SemaphoreType.DMA,
            pltpu.SemaphoreType.DMA,
        ],
    )(_sc_body)


def _tc_sum_body(x_ref, out_ref, acc_ref):
    i = pl.program_id(0)

    @pl.when(i == 0)
    def _init():
        acc_ref[0, 0] = 0.0

    acc_ref[0, 0] += jnp.sum(x_ref[...])

    @pl.when(i == _TC_GRID - 1)
    def _fin():
        out_ref[0, 0] = acc_ref[0, 0]


def _combine_body(s_ref, p_ref, out_ref):
    g = jnp.sum(p_ref[0])
    ssc = jnp.sum(p_ref[1])
    out_ref[0, 0] = _K0 + _K1 * (s_ref[0, 0] + ssc) + _K2 * g


def kernel(X, target):
    xt = X.T
    sc_parts = _sc_call()(xt, target)
    tc_sum = pl.pallas_call(
        _tc_sum_body,
        grid=(_TC_GRID,),
        in_specs=[pl.BlockSpec((_TC_BLK, _B), lambda i: (i + _TC_OFF, 0))],
        out_specs=pl.BlockSpec(
            (1, 1), lambda i: (0, 0), memory_space=pltpu.SMEM
        ),
        out_shape=jax.ShapeDtypeStruct((1, 1), jnp.float32),
        scratch_shapes=[pltpu.SMEM((1, 1), jnp.float32)],
    )(xt)
    out = pl.pallas_call(
        _combine_body,
        in_specs=[
            pl.BlockSpec(memory_space=pltpu.SMEM),
            pl.BlockSpec((2, _NW, _NL), lambda: (0, 0, 0)),
        ],
        out_specs=pl.BlockSpec(memory_space=pltpu.SMEM),
        out_shape=jax.ShapeDtypeStruct((1, 1), jnp.float32),
    )(tc_sum, sc_parts)
    return out.reshape(())
